# baseline (speedup 1.0000x reference)
import jax
import jax.numpy as jnp
import numpy as np
from jax import lax
from jax.experimental import pallas as pl
from jax.experimental.pallas import tpu as pltpu
from jax.experimental.pallas import tpu_sc as plsc

NUM_TYPES = 128
N_NODES = 100000

_NW = 32
_SPAN = 1600
_CHUNK = 400
_NCHUNK = _SPAN // _CHUNK
_GROUPS = _CHUNK // 16
_CBYTES = _CHUNK * NUM_TYPES

_TC_BLOCK = 20480


def _sc_body(types_hbm, zeros_hbm, out_hbm, types_v, buf_a, buf_b, sem):
    wid = lax.axis_index("s") * 2 + lax.axis_index("c")
    base_a = wid * _SPAN
    base_b = jnp.minimum((wid + _NW) * _SPAN, N_NODES - _SPAN)
    pltpu.sync_copy(types_hbm.at[pl.ds(base_a, _SPAN)], types_v.at[pl.ds(0, _SPAN)])
    pltpu.sync_copy(types_hbm.at[pl.ds(base_b, _SPAN)], types_v.at[pl.ds(_SPAN, _SPAN)])
    pltpu.sync_copy(zeros_hbm, buf_a)
    pltpu.sync_copy(zeros_hbm, buf_b)
    ones16 = jnp.ones((16,), jnp.float32)
    zeros16 = jnp.zeros((16,), jnp.float32)
    rows128 = lax.iota(jnp.int32, 16) * NUM_TYPES

    def scat(toff, c, buf, val):
        for g in range(_GROUPS):
            t = types_v[pl.ds(toff + c * _CHUNK + g * 16, 16)]
            plsc.store_scatter(buf, [rows128 + (g * 16 * NUM_TYPES) + t], val)

    for c in range(_NCHUNK):
        scat(0, c, buf_a, ones16)
        scat(_SPAN, c, buf_b, ones16)
        cp_a = pltpu.async_copy(
            buf_a, out_hbm.at[pl.ds((base_a + c * _CHUNK) * NUM_TYPES, _CBYTES)], sem
        )
        cp_b = pltpu.async_copy(
            buf_b, out_hbm.at[pl.ds((base_b + c * _CHUNK) * NUM_TYPES, _CBYTES)], sem
        )
        cp_a.wait()
        cp_b.wait()
        if c + 1 < _NCHUNK:
            scat(0, c, buf_a, zeros16)
            scat(_SPAN, c, buf_b, zeros16)


def _tc_body(types_ref, out_ref):
    t = types_ref[...]
    cols = jax.lax.broadcasted_iota(jnp.int32, (_TC_BLOCK, NUM_TYPES), 1)
    out_ref[...] = (cols == t[:, None]).astype(jnp.float32)


_ZEROS = np.zeros((_CBYTES,), np.float32)


def kernel(node_types, pos):
    types_flat = jnp.reshape(node_types, (N_NODES,))
    mesh = plsc.VectorSubcoreMesh(core_axis_name="c", subcore_axis_name="s")
    sc_k = pl.kernel(
        _sc_body,
        out_type=jax.ShapeDtypeStruct((N_NODES * NUM_TYPES,), jnp.float32),
        mesh=mesh,
        compiler_params=pltpu.CompilerParams(needs_layout_passes=False),
        scratch_types=[
            pltpu.VMEM((2 * _SPAN,), jnp.int32),
            pltpu.VMEM((_CBYTES,), jnp.float32),
            pltpu.VMEM((_CBYTES,), jnp.float32),
            pltpu.SemaphoreType.DMA,
        ],
    )
    out0 = jnp.reshape(sc_k(types_flat, _ZEROS), (N_NODES, NUM_TYPES))

    out1 = pl.pallas_call(
        _tc_body,
        grid=(pl.cdiv(N_NODES, _TC_BLOCK),),
        in_specs=[pl.BlockSpec((_TC_BLOCK,), lambda i: (i,))],
        out_specs=pl.BlockSpec((_TC_BLOCK, NUM_TYPES), lambda i: (i, 0)),
        out_shape=jax.ShapeDtypeStruct((N_NODES, NUM_TYPES), jnp.float32),
    )(types_flat)
    return (out0, out1)


# speedup vs baseline: 1.1855x; 1.1855x over previous
import jax
import jax.numpy as jnp
import numpy as np
from jax import lax
from jax.experimental import pallas as pl
from jax.experimental.pallas import tpu as pltpu
from jax.experimental.pallas import tpu_sc as plsc

NUM_TYPES = 128
N_NODES = 100000

_SPAN = 3200
_CHUNK = 800
_NCHUNK = _SPAN // _CHUNK
_GROUPS = _CHUNK // 16


def _sc_body(types_hbm, zeros_hbm, out0_hbm, out1_hbm, types_v, buf, sem):
    wid = lax.axis_index("s") * 2 + lax.axis_index("c")
    base = jnp.minimum(wid * _SPAN, N_NODES - _SPAN)
    pltpu.sync_copy(types_hbm.at[pl.ds(base, _SPAN)], types_v)
    pltpu.sync_copy(zeros_hbm, buf)
    ones16 = jnp.ones((16,), jnp.float32)
    zeros16 = jnp.zeros((16,), jnp.float32)
    iota16 = lax.iota(jnp.int32, 16)

    def do_chunk(c, _):
        def scat(g, _):
            t = types_v[pl.ds(c * _CHUNK + g * 16, 16)]
            plsc.store_scatter(buf, [(g * 16 + iota16) * NUM_TYPES + t], ones16)
            return 0

        lax.fori_loop(0, _GROUPS, scat, 0)
        flat0 = (base + c * _CHUNK) * NUM_TYPES
        cp0 = pltpu.async_copy(buf, out0_hbm.at[pl.ds(flat0, _CHUNK * NUM_TYPES)], sem)
        cp1 = pltpu.async_copy(buf, out1_hbm.at[pl.ds(flat0, _CHUNK * NUM_TYPES)], sem)
        cp0.wait()
        cp1.wait()

        def unscat(g, _):
            t = types_v[pl.ds(c * _CHUNK + g * 16, 16)]
            plsc.store_scatter(buf, [(g * 16 + iota16) * NUM_TYPES + t], zeros16)
            return 0

        lax.fori_loop(0, _GROUPS, unscat, 0)
        return 0

    lax.fori_loop(0, _NCHUNK, do_chunk, 0)


_ZEROS = np.zeros((_CHUNK * NUM_TYPES,), np.float32)


def kernel(node_types, pos):
    types = jnp.reshape(node_types, (N_NODES,))
    mesh = plsc.VectorSubcoreMesh(core_axis_name="c", subcore_axis_name="s")
    k = pl.kernel(
        _sc_body,
        out_type=[
            jax.ShapeDtypeStruct((N_NODES * NUM_TYPES,), jnp.float32),
            jax.ShapeDtypeStruct((N_NODES * NUM_TYPES,), jnp.float32),
        ],
        mesh=mesh,
        compiler_params=pltpu.CompilerParams(needs_layout_passes=False),
        scratch_types=[
            pltpu.VMEM((_SPAN,), jnp.int32),
            pltpu.VMEM((_CHUNK * NUM_TYPES,), jnp.float32),
            pltpu.SemaphoreType.DMA,
        ],
    )
    out0, out1 = k(types, _ZEROS)
    shape = (N_NODES, NUM_TYPES)
    return (jnp.reshape(out0, shape), jnp.reshape(out1, shape))
